# baseline (speedup 1.0000x reference)
import functools

import jax
import jax.numpy as jnp
from jax import lax
from jax.experimental import pallas as pl
from jax.experimental.pallas import tpu as pltpu
from jax.experimental.pallas import tpu_sc as plsc

_N = 10000
_E = 320000
_D = 128
_B = 64

_NC = 2
_NS = 16
_NW = _NC * _NS
_CH = 125
_NGROUP = 5
_GC = 16
_NCHUNK = _NGROUP * _GC
_EPT = _CH * _NCHUNK
_NPAD = 10240
_RPT = _NPAD // _NS


@functools.cache
def _make_sc_edge_aggr():
    mesh = plsc.VectorSubcoreMesh(core_axis_name="c", subcore_axis_name="s")

    @functools.partial(
        pl.kernel,
        mesh=mesh,
        out_type=jax.ShapeDtypeStruct((_NC, _NPAD, _D), jnp.float32),
        scratch_types=[
            pltpu.VMEM((_GC, _CH), jnp.int32),
            pltpu.VMEM((_GC, _CH), jnp.int32),
            pltpu.VMEM((_CH, _D), jnp.float32),
            pltpu.VMEM((_CH, _D), jnp.float32),
            pltpu.VMEM_SHARED((_NPAD, _D), jnp.float32),
            pltpu.SemaphoreType.DMA,
            pltpu.SemaphoreType.DMA,
            pltpu.SemaphoreType.DMA,
            pltpu.SemaphoreType.DMA,
            pltpu.SemaphoreType.DMA,
        ],
    )
    def _sc_edge_aggr(src_hbm, dst_hbm, feat_hbm, zeros_hbm, out_hbm,
                      src_v, dst_v, rows0_v, rows1_v, aggr_sh,
                      semg0, semg1, sems0, sems1, semi):
        c = lax.axis_index("c")
        s = lax.axis_index("s")
        wid = c * _NS + s
        isrc = pltpu.async_copy(src_hbm.at[wid, 0], src_v, semi)
        idst = pltpu.async_copy(dst_hbm.at[wid, 0], dst_v, semi)
        pltpu.sync_copy(zeros_hbm.at[pl.ds(s * _RPT, _RPT)],
                        aggr_sh.at[pl.ds(s * _RPT, _RPT)])
        isrc.wait()
        idst.wait()

        rows = (rows0_v, rows1_v)
        gsems = (semg0, semg1)
        ssems = (sems0, sems1)

        def gather(j, slot):
            pltpu.async_copy(feat_hbm.at[src_v.at[j]], rows[slot], gsems[slot])

        def wait_gather(slot):
            pltpu.make_async_copy(feat_hbm.at[src_v.at[0]], rows[slot],
                                  gsems[slot]).wait()

        def scatter(j, slot):
            pltpu.async_copy(rows[slot], aggr_sh.at[dst_v.at[j]], ssems[slot],
                             add=True)

        def wait_scatter(slot):
            pltpu.make_async_copy(rows[slot], aggr_sh.at[dst_v.at[0]],
                                  ssems[slot]).wait()

        gather(0, 0)
        gather(1, 1)
        plsc.subcore_barrier()

        for g in range(_NGROUP):
            if g > 0:
                pltpu.sync_copy(src_hbm.at[wid, g], src_v)
                pltpu.sync_copy(dst_hbm.at[wid, g], dst_v)
                gather(0, 0)
                gather(1, 1)

            def body(i, carry):
                j = 2 * i
                wait_gather(0)
                scatter(j, 0)
                wait_gather(1)
                scatter(j + 1, 1)
                wait_scatter(0)
                gather(j + 2, 0)
                wait_scatter(1)
                gather(j + 3, 1)
                return carry

            lax.fori_loop(0, _GC // 2 - 1, body, 0)
            wait_gather(0)
            scatter(_GC - 2, 0)
            wait_gather(1)
            scatter(_GC - 1, 1)
            wait_scatter(0)
            wait_scatter(1)

        plsc.subcore_barrier()
        pltpu.sync_copy(aggr_sh.at[pl.ds(s * _RPT, _RPT)],
                        out_hbm.at[c, pl.ds(s * _RPT, _RPT)])

    return _sc_edge_aggr


_ROWS = 1000
_NBLK = _N // _ROWS


def _lin0_body(x_ref, w_ref, b_ref, o_ref):
    o_ref[...] = jax.nn.relu(
        jnp.dot(x_ref[...], w_ref[...], preferred_element_type=jnp.float32)
        + b_ref[...])


def _lin0(x, w0t, b0r):
    return pl.pallas_call(
        _lin0_body,
        grid=(_NBLK,),
        in_specs=[
            pl.BlockSpec((_ROWS, _D), lambda i: (i, 0)),
            pl.BlockSpec((_D, _D), lambda i: (0, 0)),
            pl.BlockSpec((1, _D), lambda i: (0, 0)),
        ],
        out_specs=pl.BlockSpec((_ROWS, _D), lambda i: (i, 0)),
        out_shape=jax.ShapeDtypeStruct((_N, _D), jnp.float32),
    )(x, w0t, b0r)


def _layer_body(h_ref, p_ref, w1_ref, c1_ref, w2_ref, c2_ref,
                wih_ref, bih_ref, whh_ref, bhh_ref, o_ref):
    h = h_ref[...]
    z = h + p_ref[0] + p_ref[1]
    t = jax.nn.relu(
        jnp.dot(z, w1_ref[...], preferred_element_type=jnp.float32)
        + c1_ref[...])
    m = jax.nn.relu(
        jnp.dot(t, w2_ref[...], preferred_element_type=jnp.float32)
        + c2_ref[...])
    gi = jnp.dot(m, wih_ref[...], preferred_element_type=jnp.float32) + bih_ref[...]
    gh = jnp.dot(h, whh_ref[...], preferred_element_type=jnp.float32) + bhh_ref[...]
    r = jax.nn.sigmoid(gi[:, :_D] + gh[:, :_D])
    zg = jax.nn.sigmoid(gi[:, _D:2 * _D] + gh[:, _D:2 * _D])
    n = jnp.tanh(gi[:, 2 * _D:] + r * gh[:, 2 * _D:])
    o_ref[...] = (1.0 - zg) * n + zg * h


def _gin_layer(h, parts, w1t, c1r, w2t, c2r, wiht, bihr, whht, bhhr):
    return pl.pallas_call(
        _layer_body,
        grid=(_NBLK,),
        in_specs=[
            pl.BlockSpec((_ROWS, _D), lambda i: (i, 0)),
            pl.BlockSpec((_NC, _ROWS, _D), lambda i: (0, i, 0)),
            pl.BlockSpec((_D, _D), lambda i: (0, 0)),
            pl.BlockSpec((1, _D), lambda i: (0, 0)),
            pl.BlockSpec((_D, _D), lambda i: (0, 0)),
            pl.BlockSpec((1, _D), lambda i: (0, 0)),
            pl.BlockSpec((_D, 3 * _D), lambda i: (0, 0)),
            pl.BlockSpec((1, 3 * _D), lambda i: (0, 0)),
            pl.BlockSpec((_D, 3 * _D), lambda i: (0, 0)),
            pl.BlockSpec((1, 3 * _D), lambda i: (0, 0)),
        ],
        out_specs=pl.BlockSpec((_ROWS, _D), lambda i: (i, 0)),
        out_shape=jax.ShapeDtypeStruct((_N, _D), jnp.float32),
    )(h, parts, w1t, c1r, w2t, c2r, wiht, bihr, whht, bhhr)


def _set2set_body(out_ref, b_ref, wih_ref, bih_ref, whh_ref, bhh_ref, q_ref):
    feats = out_ref[...]
    ids = b_ref[...]
    cols = lax.broadcasted_iota(jnp.int32, (_N, _B), 1)
    maskf = jnp.where(ids == cols, 1.0, 0.0)

    qh = jnp.zeros((_B, _D), dtype=jnp.float32)
    qc = jnp.zeros((_B, _D), dtype=jnp.float32)
    q_star = jnp.zeros((_B, 2 * _D), dtype=jnp.float32)
    for _ in range(3):
        gates = (jnp.dot(q_star, wih_ref[...], preferred_element_type=jnp.float32)
                 + bih_ref[...]
                 + jnp.dot(qh, whh_ref[...], preferred_element_type=jnp.float32)
                 + bhh_ref[...])
        ig = jax.nn.sigmoid(gates[:, :_D])
        fg = jax.nn.sigmoid(gates[:, _D:2 * _D])
        gg = jnp.tanh(gates[:, 2 * _D:3 * _D])
        og = jax.nn.sigmoid(gates[:, 3 * _D:])
        qc = fg * qc + ig * gg
        qh = og * jnp.tanh(qc)

        scores = lax.dot_general(feats, qh, (((1,), (1,)), ((), ())),
                                 preferred_element_type=jnp.float32)
        e = jnp.sum(scores * maskf, axis=1, keepdims=True)
        emasked = jnp.where(maskf > 0.0, e, -jnp.inf)
        emax = jnp.max(emasked, axis=0, keepdims=True)
        emax = jnp.where(emax > -1e30, emax, 0.0)
        gmax = jnp.sum(maskf * emax, axis=1, keepdims=True)
        ex = jnp.exp(e - gmax)
        denom = jnp.sum(maskf * ex, axis=0, keepdims=True)
        gden = jnp.sum(maskf * denom, axis=1, keepdims=True)
        a = ex / (gden + 1e-16)
        r = lax.dot_general(maskf * a, feats, (((0,), (0,)), ((), ())),
                            preferred_element_type=jnp.float32)
        q_star = jnp.concatenate([qh, r], axis=1)
    q_ref[...] = q_star


def _set2set(feats, batch2d, wiht, bihr, whht, bhhr):
    return pl.pallas_call(
        _set2set_body,
        out_shape=jax.ShapeDtypeStruct((_B, 2 * _D), jnp.float32),
    )(feats, batch2d, wiht, bihr, whht, bhhr)


def kernel(x, edge_index, batch, W0, b0, gru_Wih, gru_Whh, gru_bih, gru_bhh,
           W1, c1, W2, c2, ls_Wih, ls_Whh, ls_bih, ls_bhh):
    src = edge_index[0].reshape(_NW, _NGROUP, _GC, _CH)
    dst = edge_index[1].reshape(_NW, _NGROUP, _GC, _CH)
    zeros = jnp.zeros((_NPAD, _D), dtype=jnp.float32)

    w0t = W0.T
    b0r = b0.reshape(1, _D)
    w1t = W1.T
    c1r = c1.reshape(1, _D)
    w2t = W2.T
    c2r = c2.reshape(1, _D)
    wiht = gru_Wih.T
    bihr = gru_bih.reshape(1, 3 * _D)
    whht = gru_Whh.T
    bhhr = gru_bhh.reshape(1, 3 * _D)
    ls_wiht = ls_Wih.T
    ls_bihr = ls_bih.reshape(1, 4 * _D)
    ls_whht = ls_Whh.T
    ls_bhhr = ls_bhh.reshape(1, 4 * _D)

    out = _lin0(x, w0t, b0r)
    for _ in range(3):
        parts = _make_sc_edge_aggr()(src, dst, out, zeros)
        out = _gin_layer(out, parts, w1t, c1r, w2t, c2r,
                         wiht, bihr, whht, bhhr)

    batch2d = batch.reshape(_N, 1)
    q_star = _set2set(out, batch2d, ls_wiht, ls_bihr, ls_whht, ls_bhhr)
    return (q_star, out)


# speedup vs baseline: 1.1821x; 1.1821x over previous
import functools

import jax
import jax.numpy as jnp
from jax import lax
from jax.experimental import pallas as pl
from jax.experimental.pallas import tpu as pltpu
from jax.experimental.pallas import tpu_sc as plsc

_N = 10000
_E = 320000
_D = 128
_B = 64

_NC = 2
_NS = 16
_NW = _NC * _NS
_CH = 125
_NGROUP = 5
_GC = 16
_NCHUNK = _NGROUP * _GC
_EPT = _CH * _NCHUNK
_NPAD = 10240
_RPT = _NPAD // _NS


@functools.cache
def _make_sc_edge_aggr():
    mesh = plsc.VectorSubcoreMesh(core_axis_name="c", subcore_axis_name="s")

    @functools.partial(
        pl.kernel,
        mesh=mesh,
        out_type=jax.ShapeDtypeStruct((_NC, _NPAD, _D), jnp.float32),
        scratch_types=[
            pltpu.VMEM((_GC, _CH), jnp.int32),
            pltpu.VMEM((_GC, _CH), jnp.int32),
            pltpu.VMEM((_CH, _D), jnp.float32),
            pltpu.VMEM((_CH, _D), jnp.float32),
            pltpu.VMEM_SHARED((_NPAD, _D), jnp.float32),
            pltpu.SemaphoreType.DMA,
            pltpu.SemaphoreType.DMA,
            pltpu.SemaphoreType.DMA,
        ],
    )
    def _sc_edge_aggr(src_hbm, dst_hbm, feat_hbm, zeros_hbm, out_hbm,
                      src_v, dst_v, rows0_v, rows1_v, aggr_sh,
                      semg0, semg1, semi):
        c = lax.axis_index("c")
        s = lax.axis_index("s")
        wid = c * _NS + s
        isrc = pltpu.async_copy(src_hbm.at[wid, 0], src_v, semi)
        idst = pltpu.async_copy(dst_hbm.at[wid, 0], dst_v, semi)
        pltpu.sync_copy(zeros_hbm.at[pl.ds(s * _RPT, _RPT)],
                        aggr_sh.at[pl.ds(s * _RPT, _RPT)])
        isrc.wait()
        idst.wait()

        rows = (rows0_v, rows1_v)
        gsems = (semg0, semg1)

        def gather(j, slot):
            pltpu.async_copy(feat_hbm.at[src_v.at[j]], rows[slot], gsems[slot])

        def wait_gather(slot):
            pltpu.make_async_copy(feat_hbm.at[src_v.at[0]], rows[slot],
                                  gsems[slot]).wait()

        def scatter(j, slot):
            pltpu.sync_copy(rows[slot], aggr_sh.at[dst_v.at[j]], add=True)

        gather(0, 0)
        gather(1, 1)
        plsc.subcore_barrier()

        for g in range(_NGROUP):
            if g > 0:
                pltpu.sync_copy(src_hbm.at[wid, g], src_v)
                pltpu.sync_copy(dst_hbm.at[wid, g], dst_v)
                gather(0, 0)
                gather(1, 1)

            def body(i, carry):
                j = 2 * i
                wait_gather(0)
                scatter(j, 0)
                gather(j + 2, 0)
                wait_gather(1)
                scatter(j + 1, 1)
                gather(j + 3, 1)
                return carry

            lax.fori_loop(0, _GC // 2 - 1, body, 0)
            wait_gather(0)
            scatter(_GC - 2, 0)
            wait_gather(1)
            scatter(_GC - 1, 1)

        plsc.subcore_barrier()
        pltpu.sync_copy(aggr_sh.at[pl.ds(s * _RPT, _RPT)],
                        out_hbm.at[c, pl.ds(s * _RPT, _RPT)])

    return _sc_edge_aggr


_ROWS = 1000
_NBLK = _N // _ROWS


def _lin0_body(x_ref, w_ref, b_ref, o_ref):
    o_ref[...] = jax.nn.relu(
        jnp.dot(x_ref[...], w_ref[...], preferred_element_type=jnp.float32)
        + b_ref[...])


def _lin0(x, w0t, b0r):
    return pl.pallas_call(
        _lin0_body,
        grid=(_NBLK,),
        in_specs=[
            pl.BlockSpec((_ROWS, _D), lambda i: (i, 0)),
            pl.BlockSpec((_D, _D), lambda i: (0, 0)),
            pl.BlockSpec((1, _D), lambda i: (0, 0)),
        ],
        out_specs=pl.BlockSpec((_ROWS, _D), lambda i: (i, 0)),
        out_shape=jax.ShapeDtypeStruct((_N, _D), jnp.float32),
    )(x, w0t, b0r)


def _layer_body(h_ref, p_ref, w1_ref, c1_ref, w2_ref, c2_ref,
                wih_ref, bih_ref, whh_ref, bhh_ref, o_ref):
    h = h_ref[...]
    z = h + p_ref[0] + p_ref[1]
    t = jax.nn.relu(
        jnp.dot(z, w1_ref[...], preferred_element_type=jnp.float32)
        + c1_ref[...])
    m = jax.nn.relu(
        jnp.dot(t, w2_ref[...], preferred_element_type=jnp.float32)
        + c2_ref[...])
    gi = jnp.dot(m, wih_ref[...], preferred_element_type=jnp.float32) + bih_ref[...]
    gh = jnp.dot(h, whh_ref[...], preferred_element_type=jnp.float32) + bhh_ref[...]
    r = jax.nn.sigmoid(gi[:, :_D] + gh[:, :_D])
    zg = jax.nn.sigmoid(gi[:, _D:2 * _D] + gh[:, _D:2 * _D])
    n = jnp.tanh(gi[:, 2 * _D:] + r * gh[:, 2 * _D:])
    o_ref[...] = (1.0 - zg) * n + zg * h


def _gin_layer(h, parts, w1t, c1r, w2t, c2r, wiht, bihr, whht, bhhr):
    return pl.pallas_call(
        _layer_body,
        grid=(_NBLK,),
        in_specs=[
            pl.BlockSpec((_ROWS, _D), lambda i: (i, 0)),
            pl.BlockSpec((_NC, _ROWS, _D), lambda i: (0, i, 0)),
            pl.BlockSpec((_D, _D), lambda i: (0, 0)),
            pl.BlockSpec((1, _D), lambda i: (0, 0)),
            pl.BlockSpec((_D, _D), lambda i: (0, 0)),
            pl.BlockSpec((1, _D), lambda i: (0, 0)),
            pl.BlockSpec((_D, 3 * _D), lambda i: (0, 0)),
            pl.BlockSpec((1, 3 * _D), lambda i: (0, 0)),
            pl.BlockSpec((_D, 3 * _D), lambda i: (0, 0)),
            pl.BlockSpec((1, 3 * _D), lambda i: (0, 0)),
        ],
        out_specs=pl.BlockSpec((_ROWS, _D), lambda i: (i, 0)),
        out_shape=jax.ShapeDtypeStruct((_N, _D), jnp.float32),
    )(h, parts, w1t, c1r, w2t, c2r, wiht, bihr, whht, bhhr)


def _set2set_body(out_ref, b_ref, wih_ref, bih_ref, whh_ref, bhh_ref, q_ref):
    feats = out_ref[...]
    ids = b_ref[...]
    cols = lax.broadcasted_iota(jnp.int32, (_N, _B), 1)
    maskf = jnp.where(ids == cols, 1.0, 0.0)

    qh = jnp.zeros((_B, _D), dtype=jnp.float32)
    qc = jnp.zeros((_B, _D), dtype=jnp.float32)
    q_star = jnp.zeros((_B, 2 * _D), dtype=jnp.float32)
    for _ in range(3):
        gates = (jnp.dot(q_star, wih_ref[...], preferred_element_type=jnp.float32)
                 + bih_ref[...]
                 + jnp.dot(qh, whh_ref[...], preferred_element_type=jnp.float32)
                 + bhh_ref[...])
        ig = jax.nn.sigmoid(gates[:, :_D])
        fg = jax.nn.sigmoid(gates[:, _D:2 * _D])
        gg = jnp.tanh(gates[:, 2 * _D:3 * _D])
        og = jax.nn.sigmoid(gates[:, 3 * _D:])
        qc = fg * qc + ig * gg
        qh = og * jnp.tanh(qc)

        scores = lax.dot_general(feats, qh, (((1,), (1,)), ((), ())),
                                 preferred_element_type=jnp.float32)
        e = jnp.sum(scores * maskf, axis=1, keepdims=True)
        emasked = jnp.where(maskf > 0.0, e, -jnp.inf)
        emax = jnp.max(emasked, axis=0, keepdims=True)
        emax = jnp.where(emax > -1e30, emax, 0.0)
        gmax = jnp.sum(maskf * emax, axis=1, keepdims=True)
        ex = jnp.exp(e - gmax)
        denom = jnp.sum(maskf * ex, axis=0, keepdims=True)
        gden = jnp.sum(maskf * denom, axis=1, keepdims=True)
        a = ex / (gden + 1e-16)
        r = lax.dot_general(maskf * a, feats, (((0,), (0,)), ((), ())),
                            preferred_element_type=jnp.float32)
        q_star = jnp.concatenate([qh, r], axis=1)
    q_ref[...] = q_star


def _set2set(feats, batch2d, wiht, bihr, whht, bhhr):
    return pl.pallas_call(
        _set2set_body,
        out_shape=jax.ShapeDtypeStruct((_B, 2 * _D), jnp.float32),
    )(feats, batch2d, wiht, bihr, whht, bhhr)


def kernel(x, edge_index, batch, W0, b0, gru_Wih, gru_Whh, gru_bih, gru_bhh,
           W1, c1, W2, c2, ls_Wih, ls_Whh, ls_bih, ls_bhh):
    src = edge_index[0].reshape(_NW, _NGROUP, _GC, _CH)
    dst = edge_index[1].reshape(_NW, _NGROUP, _GC, _CH)
    zeros = jnp.zeros((_NPAD, _D), dtype=jnp.float32)

    w0t = W0.T
    b0r = b0.reshape(1, _D)
    w1t = W1.T
    c1r = c1.reshape(1, _D)
    w2t = W2.T
    c2r = c2.reshape(1, _D)
    wiht = gru_Wih.T
    bihr = gru_bih.reshape(1, 3 * _D)
    whht = gru_Whh.T
    bhhr = gru_bhh.reshape(1, 3 * _D)
    ls_wiht = ls_Wih.T
    ls_bihr = ls_bih.reshape(1, 4 * _D)
    ls_whht = ls_Whh.T
    ls_bhhr = ls_bhh.reshape(1, 4 * _D)

    out = _lin0(x, w0t, b0r)
    for _ in range(3):
        parts = _make_sc_edge_aggr()(src, dst, out, zeros)
        out = _gin_layer(out, parts, w1t, c1r, w2t, c2r,
                         wiht, bihr, whht, bhhr)

    batch2d = batch.reshape(_N, 1)
    q_star = _set2set(out, batch2d, ls_wiht, ls_bihr, ls_whht, ls_bhhr)
    return (q_star, out)


# speedup vs baseline: 1.2645x; 1.0697x over previous
import functools

import jax
import jax.numpy as jnp
from jax import lax
from jax.experimental import pallas as pl
from jax.experimental.pallas import tpu as pltpu
from jax.experimental.pallas import tpu_sc as plsc

_N = 10000
_E = 320000
_D = 128
_B = 64

_NC = 2
_NS = 16
_NW = _NC * _NS
_CH = 125
_NGROUP = 4
_GC = 20
_NCHUNK = _NGROUP * _GC
_EPT = _CH * _NCHUNK
_NPAD = 10240
_RPT = _NPAD // _NS


@functools.cache
def _make_sc_edge_aggr():
    mesh = plsc.VectorSubcoreMesh(core_axis_name="c", subcore_axis_name="s")

    @functools.partial(
        pl.kernel,
        mesh=mesh,
        out_type=jax.ShapeDtypeStruct((_NC, _NPAD, _D), jnp.float32),
        scratch_types=[
            pltpu.VMEM((2, _GC, _CH), jnp.int32),
            pltpu.VMEM((2, _GC, _CH), jnp.int32),
            pltpu.VMEM((_CH, _D), jnp.float32),
            pltpu.VMEM((_CH, _D), jnp.float32),
            pltpu.VMEM_SHARED((_NPAD, _D), jnp.float32),
            pltpu.SemaphoreType.DMA,
            pltpu.SemaphoreType.DMA,
            pltpu.SemaphoreType.DMA,
        ],
    )
    def _sc_edge_aggr(src_hbm, dst_hbm, feat_hbm, zeros_hbm, out_hbm,
                      src_v, dst_v, rows0_v, rows1_v, aggr_sh,
                      semg0, semg1, semi):
        c = lax.axis_index("c")
        s = lax.axis_index("s")
        wid = c * _NS + s
        isrc = pltpu.async_copy(src_hbm.at[wid, 0], src_v.at[0], semi)
        idst = pltpu.async_copy(dst_hbm.at[wid, 0], dst_v.at[0], semi)
        pltpu.sync_copy(zeros_hbm.at[pl.ds(s * _RPT, _RPT)],
                        aggr_sh.at[pl.ds(s * _RPT, _RPT)])
        isrc.wait()
        idst.wait()

        rows = (rows0_v, rows1_v)
        gsems = (semg0, semg1)

        def gather(sl, j, slot):
            pltpu.async_copy(feat_hbm.at[src_v.at[sl, j]], rows[slot],
                             gsems[slot])

        def wait_gather(slot):
            pltpu.make_async_copy(feat_hbm.at[src_v.at[0, 0]], rows[slot],
                                  gsems[slot]).wait()

        def scatter(sl, j, slot):
            pltpu.sync_copy(rows[slot], aggr_sh.at[dst_v.at[sl, j]], add=True)

        gather(0, 0, 0)
        gather(0, 1, 1)
        plsc.subcore_barrier()

        for g in range(_NGROUP):
            sl = g % 2
            nxt = (g + 1) % 2
            if g + 1 < _NGROUP:
                psrc = pltpu.async_copy(src_hbm.at[wid, g + 1],
                                        src_v.at[nxt], semi)
                pdst = pltpu.async_copy(dst_hbm.at[wid, g + 1],
                                        dst_v.at[nxt], semi)

            def body(i, carry, sl=sl):
                j = 2 * i
                wait_gather(0)
                scatter(sl, j, 0)
                gather(sl, j + 2, 0)
                wait_gather(1)
                scatter(sl, j + 1, 1)
                gather(sl, j + 3, 1)
                return carry

            lax.fori_loop(0, _GC // 2 - 1, body, 0)
            wait_gather(0)
            scatter(sl, _GC - 2, 0)
            wait_gather(1)
            scatter(sl, _GC - 1, 1)
            if g + 1 < _NGROUP:
                psrc.wait()
                pdst.wait()
                gather(nxt, 0, 0)
                gather(nxt, 1, 1)

        plsc.subcore_barrier()
        pltpu.sync_copy(aggr_sh.at[pl.ds(s * _RPT, _RPT)],
                        out_hbm.at[c, pl.ds(s * _RPT, _RPT)])

    return _sc_edge_aggr


_ROWS = 1000
_NBLK = _N // _ROWS


def _lin0_body(x_ref, w_ref, b_ref, o_ref):
    o_ref[...] = jax.nn.relu(
        jnp.dot(x_ref[...], w_ref[...], preferred_element_type=jnp.float32)
        + b_ref[...])


def _lin0(x, w0t, b0r):
    return pl.pallas_call(
        _lin0_body,
        grid=(_NBLK,),
        in_specs=[
            pl.BlockSpec((_ROWS, _D), lambda i: (i, 0)),
            pl.BlockSpec((_D, _D), lambda i: (0, 0)),
            pl.BlockSpec((1, _D), lambda i: (0, 0)),
        ],
        out_specs=pl.BlockSpec((_ROWS, _D), lambda i: (i, 0)),
        out_shape=jax.ShapeDtypeStruct((_N, _D), jnp.float32),
    )(x, w0t, b0r)


def _layer_body(h_ref, p_ref, w1_ref, c1_ref, w2_ref, c2_ref,
                wih_ref, bih_ref, whh_ref, bhh_ref, o_ref):
    h = h_ref[...]
    z = h + p_ref[0] + p_ref[1]
    t = jax.nn.relu(
        jnp.dot(z, w1_ref[...], preferred_element_type=jnp.float32)
        + c1_ref[...])
    m = jax.nn.relu(
        jnp.dot(t, w2_ref[...], preferred_element_type=jnp.float32)
        + c2_ref[...])
    gi = jnp.dot(m, wih_ref[...], preferred_element_type=jnp.float32) + bih_ref[...]
    gh = jnp.dot(h, whh_ref[...], preferred_element_type=jnp.float32) + bhh_ref[...]
    r = jax.nn.sigmoid(gi[:, :_D] + gh[:, :_D])
    zg = jax.nn.sigmoid(gi[:, _D:2 * _D] + gh[:, _D:2 * _D])
    n = jnp.tanh(gi[:, 2 * _D:] + r * gh[:, 2 * _D:])
    o_ref[...] = (1.0 - zg) * n + zg * h


def _gin_layer(h, parts, w1t, c1r, w2t, c2r, wiht, bihr, whht, bhhr):
    return pl.pallas_call(
        _layer_body,
        grid=(_NBLK,),
        in_specs=[
            pl.BlockSpec((_ROWS, _D), lambda i: (i, 0)),
            pl.BlockSpec((_NC, _ROWS, _D), lambda i: (0, i, 0)),
            pl.BlockSpec((_D, _D), lambda i: (0, 0)),
            pl.BlockSpec((1, _D), lambda i: (0, 0)),
            pl.BlockSpec((_D, _D), lambda i: (0, 0)),
            pl.BlockSpec((1, _D), lambda i: (0, 0)),
            pl.BlockSpec((_D, 3 * _D), lambda i: (0, 0)),
            pl.BlockSpec((1, 3 * _D), lambda i: (0, 0)),
            pl.BlockSpec((_D, 3 * _D), lambda i: (0, 0)),
            pl.BlockSpec((1, 3 * _D), lambda i: (0, 0)),
        ],
        out_specs=pl.BlockSpec((_ROWS, _D), lambda i: (i, 0)),
        out_shape=jax.ShapeDtypeStruct((_N, _D), jnp.float32),
    )(h, parts, w1t, c1r, w2t, c2r, wiht, bihr, whht, bhhr)


def _set2set_body(out_ref, b_ref, wih_ref, bih_ref, whh_ref, bhh_ref, q_ref):
    feats = out_ref[...]
    ids = b_ref[...]
    cols = lax.broadcasted_iota(jnp.int32, (_N, _B), 1)
    maskf = jnp.where(ids == cols, 1.0, 0.0)

    qh = jnp.zeros((_B, _D), dtype=jnp.float32)
    qc = jnp.zeros((_B, _D), dtype=jnp.float32)
    q_star = jnp.zeros((_B, 2 * _D), dtype=jnp.float32)
    for _ in range(3):
        gates = (jnp.dot(q_star, wih_ref[...], preferred_element_type=jnp.float32)
                 + bih_ref[...]
                 + jnp.dot(qh, whh_ref[...], preferred_element_type=jnp.float32)
                 + bhh_ref[...])
        ig = jax.nn.sigmoid(gates[:, :_D])
        fg = jax.nn.sigmoid(gates[:, _D:2 * _D])
        gg = jnp.tanh(gates[:, 2 * _D:3 * _D])
        og = jax.nn.sigmoid(gates[:, 3 * _D:])
        qc = fg * qc + ig * gg
        qh = og * jnp.tanh(qc)

        scores = lax.dot_general(feats, qh, (((1,), (1,)), ((), ())),
                                 preferred_element_type=jnp.float32)
        e = jnp.sum(scores * maskf, axis=1, keepdims=True)
        emasked = jnp.where(maskf > 0.0, e, -jnp.inf)
        emax = jnp.max(emasked, axis=0, keepdims=True)
        emax = jnp.where(emax > -1e30, emax, 0.0)
        gmax = jnp.sum(maskf * emax, axis=1, keepdims=True)
        ex = jnp.exp(e - gmax)
        denom = jnp.sum(maskf * ex, axis=0, keepdims=True)
        gden = jnp.sum(maskf * denom, axis=1, keepdims=True)
        a = ex / (gden + 1e-16)
        r = lax.dot_general(maskf * a, feats, (((0,), (0,)), ((), ())),
                            preferred_element_type=jnp.float32)
        q_star = jnp.concatenate([qh, r], axis=1)
    q_ref[...] = q_star


def _set2set(feats, batch2d, wiht, bihr, whht, bhhr):
    return pl.pallas_call(
        _set2set_body,
        out_shape=jax.ShapeDtypeStruct((_B, 2 * _D), jnp.float32),
    )(feats, batch2d, wiht, bihr, whht, bhhr)


def kernel(x, edge_index, batch, W0, b0, gru_Wih, gru_Whh, gru_bih, gru_bhh,
           W1, c1, W2, c2, ls_Wih, ls_Whh, ls_bih, ls_bhh):
    src = edge_index[0].reshape(_NW, _NGROUP, _GC, _CH)
    dst = edge_index[1].reshape(_NW, _NGROUP, _GC, _CH)
    zeros = jnp.zeros((_NPAD, _D), dtype=jnp.float32)

    w0t = W0.T
    b0r = b0.reshape(1, _D)
    w1t = W1.T
    c1r = c1.reshape(1, _D)
    w2t = W2.T
    c2r = c2.reshape(1, _D)
    wiht = gru_Wih.T
    bihr = gru_bih.reshape(1, 3 * _D)
    whht = gru_Whh.T
    bhhr = gru_bhh.reshape(1, 3 * _D)
    ls_wiht = ls_Wih.T
    ls_bihr = ls_bih.reshape(1, 4 * _D)
    ls_whht = ls_Whh.T
    ls_bhhr = ls_bhh.reshape(1, 4 * _D)

    out = _lin0(x, w0t, b0r)
    for _ in range(3):
        parts = _make_sc_edge_aggr()(src, dst, out, zeros)
        out = _gin_layer(out, parts, w1t, c1r, w2t, c2r,
                         wiht, bihr, whht, bhhr)

    batch2d = batch.reshape(_N, 1)
    q_star = _set2set(out, batch2d, ls_wiht, ls_bihr, ls_whht, ls_bhhr)
    return (q_star, out)
